# Initial kernel scaffold; baseline (speedup 1.0000x reference)
import jax
import jax.numpy as jnp
from jax.experimental import pallas as pl


def kernel(agents, my_id, emb_table, agent_w, agent_b, my_w, my_b):
    raise NotImplementedError("write your pallas kernel here")


# speedup vs baseline: 68.1854x; 68.1854x over previous
import functools

import jax
import jax.numpy as jnp
from jax import lax
from jax.experimental import pallas as pl
from jax.experimental.pallas import tpu as pltpu
from jax.experimental.pallas import tpu_sc as plsc

_B = 1024
_A = 100
_ATTRS = 23
_EMB = 32
_FAN = _ATTRS * _EMB
_R = _B * _A
_R23 = _R * _ATTRS
_NC, _NS = 2, 16
_NW = _NC * _NS
_EW = _R23 // _NW
_STREAM = 128
_CH = 5
_CHUNK = _CH * _STREAM
_NCHUNK = _EW // _CHUNK
_IDXROWS = _R23 // _STREAM
_ROWS_W = _IDXROWS // _NW

_mesh = functools.partial(plsc.VectorSubcoreMesh,
                          core_axis_name="c", subcore_axis_name="s")


def _wid():
    return lax.axis_index("s") * _NC + lax.axis_index("c")


@functools.partial(
    pl.kernel,
    mesh=_mesh(),
    compiler_params=pltpu.CompilerParams(use_tc_tiling_on_sc=False),
    out_type=jax.ShapeDtypeStruct((_R23, _EMB), jnp.float32),
    scratch_types=[
        pltpu.VMEM((_CHUNK,), jnp.int32),
        pltpu.VMEM((_CHUNK, _EMB), jnp.float32),
        pltpu.SemaphoreType.DMA,
    ],
)
def _sc_gather(idx_hbm, table_hbm, out_hbm, idx_v, rows_v, sem):
    base = _wid() * _EW

    def body(ci, carry):
        off = base + ci * _CHUNK
        pltpu.sync_copy(idx_hbm.at[pl.ds(off, _CHUNK)], idx_v)
        copies = [
            pltpu.async_copy(
                table_hbm.at[idx_v.at[pl.ds(j * _STREAM, _STREAM)]],
                rows_v.at[pl.ds(j * _STREAM, _STREAM)],
                sem,
            )
            for j in range(_CH)
        ]
        for c in copies:
            c.wait()
        pltpu.sync_copy(rows_v, out_hbm.at[pl.ds(off, _CHUNK)])
        return carry

    lax.fori_loop(0, _NCHUNK, body, 0)


_BW = _B // _NW


@functools.partial(
    pl.kernel,
    mesh=_mesh(),
    compiler_params=pltpu.CompilerParams(use_tc_tiling_on_sc=False),
    out_type=jax.ShapeDtypeStruct((_B, _FAN), jnp.float32),
    scratch_types=[
        pltpu.VMEM((_BW,), jnp.int32),
        pltpu.VMEM((_BW, _FAN), jnp.float32),
        pltpu.SemaphoreType.DMA,
    ],
)
def _sc_my_gather(g_hbm, emb_hbm, out_hbm, g_v, rows_v, sem):
    base = _wid() * _BW
    pltpu.sync_copy(g_hbm.at[pl.ds(base, _BW)], g_v)
    pltpu.async_copy(emb_hbm.at[g_v], rows_v, sem).wait()
    pltpu.sync_copy(rows_v, out_hbm.at[pl.ds(base, _BW)])


def _rowsel_body(ids_ref, my_ref, g_ref):
    ids = ids_ref[...]
    match = (ids == my_ref[...]) & (ids != 0)
    lane = lax.broadcasted_iota(jnp.int32, ids.shape, 1)
    cand = jnp.where(match, lane, 16384)
    row = jnp.min(cand, axis=1, keepdims=True)
    row = jnp.where(row >= 16384, 0, row)
    bidx = lax.broadcasted_iota(jnp.int32, row.shape, 0)
    g_ref[...] = jnp.broadcast_to(bidx * _A + row, ids.shape)


def _rowsel(ids_pad, my2):
    return pl.pallas_call(
        _rowsel_body,
        out_shape=jax.ShapeDtypeStruct((_B, 128), jnp.int32),
    )(ids_pad, my2)


_MBLK = 1024


def _mm_body(x_ref, w_ref, b_ref, o_ref):
    o_ref[...] = (
        jnp.dot(x_ref[...], w_ref[...], preferred_element_type=jnp.float32)
        + b_ref[...]
    )


def _mm(emb, wt, bias):
    return pl.pallas_call(
        _mm_body,
        grid=(_R // _MBLK,),
        in_specs=[
            pl.BlockSpec((_MBLK, _FAN), lambda i: (i, 0)),
            pl.BlockSpec((_FAN, 512), lambda i: (0, 0)),
            pl.BlockSpec((1, 512), lambda i: (0, 0)),
        ],
        out_specs=pl.BlockSpec((_MBLK, 512), lambda i: (i, 0)),
        out_shape=jax.ShapeDtypeStruct((_R, 512), jnp.float32),
    )(emb, wt, bias)


def _myfc_body(x_ref, w_ref, b_ref, o_ref):
    o_ref[...] = jnp.maximum(
        jnp.dot(x_ref[...], w_ref[...], preferred_element_type=jnp.float32)
        + b_ref[...],
        0.0,
    )


def _myfc(x, wt, bias):
    return pl.pallas_call(
        _myfc_body,
        out_shape=jax.ShapeDtypeStruct((_B, 512), jnp.float32),
    )(x, wt, bias)


def kernel(agents, my_id, emb_table, agent_w, agent_b, my_w, my_b):
    idx = jnp.clip(agents, 0, 255) + jnp.arange(_ATTRS, dtype=jnp.int32) * 256
    emb = _sc_gather(idx.reshape(_R23), emb_table).reshape(_R, _FAN)

    ids_pad = jnp.pad(agents[:, :, 0], ((0, 0), (0, 128 - _A)))
    my2 = jnp.broadcast_to(my_id[:, None], (_B, 128))
    g = _rowsel(ids_pad, my2)[:, 0]

    my_emb = _sc_my_gather(g, emb)

    agent_out = _mm(emb, agent_w.T, agent_b[None, :]).reshape(_B, _A, 512)
    my_out = _myfc(my_emb, my_w.T, my_b[None, :])
    return agent_out, my_out
